# baseline (speedup 1.0000x reference)
import functools

import jax
import jax.numpy as jnp
from jax import lax
from jax.experimental import pallas as pl
from jax.experimental.pallas import tpu as pltpu
from jax.experimental.pallas import tpu_sc as plsc

EPS = 1e-12
NW = 32
H = 128

TOK_K, TOK_NC = 80, 80
ET_K, ET_NC = 80, 200
ET_SEG = 5
EE_K, EE_NC = 80, 20
CD_K, CD_NC = 80, 40


def _gather_task(wid, table, idx_hbm, idx_buf, out, bufs, sgs, sws,
                 nchunks, k, rpw):
    pltpu.sync_copy(idx_hbm.at[wid], idx_buf)
    base = wid * rpw

    def g_start(c, i):
        pltpu.make_async_copy(table.at[idx_buf.at[c]], bufs[i], sgs[i]).start()

    def g_wait(c, i):
        pltpu.make_async_copy(table.at[idx_buf.at[c]], bufs[i], sgs[i]).wait()

    def w_start(c, i):
        pltpu.make_async_copy(bufs[i], out.at[pl.ds(base + c * k, k)],
                              sws[i]).start()

    def w_wait(c, i):
        pltpu.make_async_copy(bufs[i], out.at[pl.ds(base + c * k, k)],
                              sws[i]).wait()

    g_start(0, 0)
    g_start(1, 1)
    g_start(2, 2)

    def quad(q, carry):
        for p in range(4):
            c = 4 * q + p
            pn = (p + 3) % 4
            g_wait(c, p)
            w_start(c, p)

            @pl.when(c > 0)
            def _(c=c, pn=pn):
                w_wait(c - 1, pn)

            @pl.when(c + 3 < nchunks)
            def _(c=c, pn=pn):
                g_start(c + 3, pn)
        return carry

    lax.fori_loop(0, nchunks // 4, quad, 0)
    w_wait(nchunks - 1, 3)


def _sc_tok_body(tok_idx, word, tok_out,
                 tok_ib, b0, b1, b2, b3, s0, s1, s2, s3, w0, w1, w2, w3):
    wid = lax.axis_index("s") * 2 + lax.axis_index("c")
    _gather_task(wid, word, tok_idx, tok_ib, tok_out, (b0, b1, b2, b3),
                 (s0, s1, s2, s3), (w0, w1, w2, w3),
                 TOK_NC, TOK_K, TOK_NC * TOK_K)


def _sc_et_body(et_idx, word, et_out,
                et_ib, b0, b1, b2, b3, acc, s0, s1, s2, s3):
    wid = lax.axis_index("s") * 2 + lax.axis_index("c")
    pltpu.sync_copy(et_idx.at[wid], et_ib)
    bufs = (b0, b1, b2, b3)
    sgs = (s0, s1, s2, s3)
    seg_chunks = ET_NC // ET_SEG
    seg_rows = seg_chunks * 8

    def g_start(c, i):
        pltpu.make_async_copy(word.at[et_ib.at[c]], bufs[i], sgs[i]).start()

    def g_wait(c, i):
        pltpu.make_async_copy(word.at[et_ib.at[c]], bufs[i], sgs[i]).wait()

    def reduce(buf, brow):
        for r in range(8):
            for cc in range(8):
                sl = pl.ds(cc * 16, 16)
                v = buf[10 * r, sl]
                for m in range(1, 10):
                    v = v + buf[10 * r + m, sl]
                acc[brow + r, sl] = v

    def seg_body(seg, carry):
        ch0 = seg * seg_chunks
        g_start(ch0 + 0, 0)
        g_start(ch0 + 1, 1)
        g_start(ch0 + 2, 2)

        def quad(q, inner):
            for p in range(4):
                lc = 4 * q + p
                g_wait(ch0 + lc, p)
                reduce(bufs[p], lc * 8)

                @pl.when(lc + 3 < seg_chunks)
                def _(lc=lc, p=p):
                    g_start(ch0 + lc + 3, (p + 3) % 4)
            return inner

        lax.fori_loop(0, seg_chunks // 4, quad, 0)
        pltpu.sync_copy(acc,
                        et_out.at[pl.ds(wid * (ET_NC * 8) + seg * seg_rows,
                                        seg_rows)])
        return carry

    lax.fori_loop(0, ET_SEG, seg_body, 0)


def _sc_cd_body(cd_idx, ee_idx, ent, cd_out, ee_out,
                cd_ib, ee_ib, b0, b1, b2, b3,
                s0, s1, s2, s3, w0, w1, w2, w3):
    wid = lax.axis_index("s") * 2 + lax.axis_index("c")
    bufs = (b0, b1, b2, b3)
    sgs = (s0, s1, s2, s3)
    sws = (w0, w1, w2, w3)
    _gather_task(wid, ent, cd_idx, cd_ib, cd_out, bufs, sgs, sws,
                 CD_NC, CD_K, CD_NC * CD_K)
    _gather_task(wid, ent, ee_idx, ee_ib, ee_out, bufs, sgs, sws,
                 EE_NC, EE_K, EE_NC * EE_K)


_MESH = dict(core_axis_name="c", subcore_axis_name="s")


def _sems(n):
    return [pltpu.SemaphoreType.DMA] * n


def _sc_tok(tok_idx, word_emb, bt):
    f32 = jnp.float32
    return pl.kernel(
        _sc_tok_body,
        out_type=jax.ShapeDtypeStruct((bt, H), f32),
        mesh=plsc.VectorSubcoreMesh(**_MESH),
        scratch_types=[pltpu.VMEM((TOK_NC, TOK_K), jnp.int32)]
        + [pltpu.VMEM((TOK_K, H), f32)] * 4 + _sems(8),
    )(tok_idx, word_emb)


def _sc_et(et_idx, word_emb, be):
    f32 = jnp.float32
    return pl.kernel(
        _sc_et_body,
        out_type=jax.ShapeDtypeStruct((be, H), f32),
        mesh=plsc.VectorSubcoreMesh(**_MESH),
        scratch_types=[pltpu.VMEM((ET_NC, ET_K), jnp.int32)]
        + [pltpu.VMEM((ET_K, H), f32)] * 4
        + [pltpu.VMEM((ET_NC // ET_SEG * 8, H), f32)] + _sems(4),
    )(et_idx, word_emb)


def _sc_cd(cd_idx, ee_idx, ent_emb, bc, be):
    f32 = jnp.float32
    return pl.kernel(
        _sc_cd_body,
        out_type=[jax.ShapeDtypeStruct((bc, H), f32),
                  jax.ShapeDtypeStruct((be, H), f32)],
        mesh=plsc.VectorSubcoreMesh(**_MESH),
        scratch_types=[pltpu.VMEM((CD_NC, CD_K), jnp.int32),
                       pltpu.VMEM((EE_NC, EE_K), jnp.int32)]
        + [pltpu.VMEM((CD_K, H), f32)] * 4 + _sems(8),
    )(cd_idx, ee_idx, ent_emb)


def _ln(x, g, b):
    m = jnp.mean(x, axis=-1, keepdims=True)
    v = jnp.mean((x - m) ** 2, axis=-1, keepdims=True)
    return (x - m) / jnp.sqrt(v + EPS) * g + b


_DN = (((0,), (0,)), ((), ()))
_DNR = (((1,), (0,)), ((), ()))


def _onehot_rows(idx_1xn, table_ref, width, blk):
    iota = lax.broadcasted_iota(jnp.int32, (width, blk), 0)
    oh = (iota == idx_1xn).astype(jnp.bfloat16)
    return lax.dot_general(oh, table_ref[...], _DN,
                           preferred_element_type=jnp.float32)


def _tok_tc_body(rows_ref, pidx_ref, tidx_ref, pos_ref, typ_ref, g_ref, b_ref,
                 out_ref, *, blk):
    x = rows_ref[...]
    x = x + _onehot_rows(pidx_ref[0], pos_ref, 256, blk)
    x = x + _onehot_rows(tidx_ref[0], typ_ref, 16, blk)
    out_ref[...] = _ln(x, g_ref[...], b_ref[...])


def _ent_tc_body(ee_ref, es_ref, lenf_ref, mnz_ref, midx_ref, tidx_ref,
                 mask_ref, w1_ref, w2_ref, fb_ref, typ_ref, g_ref, b_ref,
                 out_ref, *, blk):
    g = g_ref[...]
    b = b_ref[...]
    et = es_ref[...] / lenf_ref[0]
    mnz = mnz_ref[0]
    mrows = _onehot_rows(midx_ref[0], mask_ref, 8, blk)
    et = mnz * mrows + (1.0 - mnz) * et
    x = lax.dot_general(ee_ref[...].astype(jnp.bfloat16), w1_ref[...], _DNR,
                        preferred_element_type=jnp.float32)
    x = x + lax.dot_general(et.astype(jnp.bfloat16), w2_ref[...], _DNR,
                            preferred_element_type=jnp.float32)
    x = x + fb_ref[...]
    x = 0.5 * x * (1.0 + lax.erf(x * (2.0 ** -0.5)))
    x = _ln(x, g, b)
    x = x + _onehot_rows(tidx_ref[0], typ_ref, 16, blk)
    out_ref[...] = _ln(x, g, b)


def kernel(input_tok, input_tok_type, input_tok_pos, input_ent_tok,
           input_ent_tok_length, input_ent_mask_type, input_ent,
           input_ent_type, ent_candidates, word_emb, ent_emb, pos_emb,
           type_emb, mask_emb, fusion_w, fusion_b, ln_g, ln_b):
    B, T = input_tok.shape
    _, E, M = input_ent_tok.shape
    _, C = ent_candidates.shape
    BT, BE, BC = B * T, B * E, B * C
    f32 = jnp.float32
    bf16 = jnp.bfloat16

    tok_idx = input_tok.reshape(NW, TOK_NC, TOK_K)
    et_idx = input_ent_tok.reshape(NW, ET_NC, ET_K)
    ee_idx = input_ent.reshape(NW, EE_NC, EE_K)
    cd_idx = ent_candidates.reshape(NW, CD_NC, CD_K)

    tok_rows = _sc_tok(tok_idx, word_emb, BT)
    et_sum = _sc_et(et_idx, word_emb, BE)
    cd_rows, ee_rows = _sc_cd(cd_idx, ee_idx, ent_emb, BC, BE)

    BLK = 512
    nb = BT // BLK
    pos256 = pos_emb[:256].astype(bf16)
    typ16 = jnp.zeros((16, H), f32).at[:type_emb.shape[0]].set(type_emb)
    typ16 = typ16.astype(bf16)
    pidx = input_tok_pos.reshape(nb, 1, BLK)
    tidx = input_tok_type.reshape(nb, 1, BLK)
    g2 = ln_g.reshape(1, H)
    b2 = ln_b.reshape(1, H)

    tok_out = pl.pallas_call(
        functools.partial(_tok_tc_body, blk=BLK),
        grid=(nb,),
        in_specs=[
            pl.BlockSpec((BLK, H), lambda i: (i, 0)),
            pl.BlockSpec((1, 1, BLK), lambda i: (i, 0, 0)),
            pl.BlockSpec((1, 1, BLK), lambda i: (i, 0, 0)),
            pl.BlockSpec((256, H), lambda i: (0, 0)),
            pl.BlockSpec((16, H), lambda i: (0, 0)),
            pl.BlockSpec((1, H), lambda i: (0, 0)),
            pl.BlockSpec((1, H), lambda i: (0, 0)),
        ],
        out_specs=pl.BlockSpec((BLK, H), lambda i: (i, 0)),
        out_shape=jax.ShapeDtypeStruct((BT, H), f32),
    )(tok_rows, pidx, tidx, pos256, typ16, g2, b2)

    BLK2 = 512
    nb2 = BE // BLK2
    lenf = input_ent_tok_length.astype(f32).reshape(nb2, BLK2, 1)
    mnz = (input_ent_mask_type != 0).astype(f32).reshape(nb2, BLK2, 1)
    midx = input_ent_mask_type.reshape(nb2, 1, BLK2)
    etidx = input_ent_type.reshape(nb2, 1, BLK2)
    mask8 = jnp.zeros((8, H), f32).at[:mask_emb.shape[0]].set(mask_emb)
    mask8 = mask8.astype(bf16)
    w1 = fusion_w[:H].astype(bf16)
    w2 = fusion_w[H:].astype(bf16)
    fb2 = fusion_b.reshape(1, H)

    ent_out = pl.pallas_call(
        functools.partial(_ent_tc_body, blk=BLK2),
        grid=(nb2,),
        in_specs=[
            pl.BlockSpec((BLK2, H), lambda i: (i, 0)),
            pl.BlockSpec((BLK2, H), lambda i: (i, 0)),
            pl.BlockSpec((1, BLK2, 1), lambda i: (i, 0, 0)),
            pl.BlockSpec((1, BLK2, 1), lambda i: (i, 0, 0)),
            pl.BlockSpec((1, 1, BLK2), lambda i: (i, 0, 0)),
            pl.BlockSpec((1, 1, BLK2), lambda i: (i, 0, 0)),
            pl.BlockSpec((8, H), lambda i: (0, 0)),
            pl.BlockSpec((H, H), lambda i: (0, 0)),
            pl.BlockSpec((H, H), lambda i: (0, 0)),
            pl.BlockSpec((1, H), lambda i: (0, 0)),
            pl.BlockSpec((16, H), lambda i: (0, 0)),
            pl.BlockSpec((1, H), lambda i: (0, 0)),
            pl.BlockSpec((1, H), lambda i: (0, 0)),
        ],
        out_specs=pl.BlockSpec((BLK2, H), lambda i: (i, 0)),
        out_shape=jax.ShapeDtypeStruct((BE, H), f32),
    )(ee_rows, et_sum, lenf, mnz, midx, etidx, mask8, w1, w2, fb2,
      typ16, g2, b2)

    return (tok_out.reshape(B, T, H),
            ent_out.reshape(B, E, H),
            cd_rows.reshape(B, C, H))


# speedup vs baseline: 1.3337x; 1.3337x over previous
import functools

import jax
import jax.numpy as jnp
from jax import lax
from jax.experimental import pallas as pl
from jax.experimental.pallas import tpu as pltpu
from jax.experimental.pallas import tpu_sc as plsc

EPS = 1e-12
NW = 32
H = 128

K = 80
TOK_NC = 80
ET_NC = 200
ET_SEG = 5
EE_NC = 20
CD_NC = 40


def _gather_task(wid, table, idx_buf, out, bufs, sgs, sws, nchunks, rpw):
    base = wid * rpw

    def g_start(c, i):
        pltpu.make_async_copy(table.at[idx_buf.at[pl.ds(c * K, K)]],
                              bufs[i], sgs[i]).start()

    def g_wait(c, i):
        pltpu.make_async_copy(table.at[idx_buf.at[pl.ds(c * K, K)]],
                              bufs[i], sgs[i]).wait()

    def w_start(c, i):
        pltpu.make_async_copy(bufs[i], out.at[pl.ds(base + c * K, K)],
                              sws[i]).start()

    def w_wait(c, i):
        pltpu.make_async_copy(bufs[i], out.at[pl.ds(base + c * K, K)],
                              sws[i]).wait()

    g_start(0, 0)
    g_start(1, 1)
    g_start(2, 2)

    def quad(q, carry):
        for p in range(4):
            c = 4 * q + p
            pn = (p + 3) % 4
            g_wait(c, p)
            w_start(c, p)

            @pl.when(c > 0)
            def _(c=c, pn=pn):
                w_wait(c - 1, pn)

            @pl.when(c + 3 < nchunks)
            def _(c=c, pn=pn):
                g_start(c + 3, pn)
        return carry

    lax.fori_loop(0, nchunks // 4, quad, 0)
    w_wait(nchunks - 1, 3)


def _sc_tok_body(tok_idx, word, tok_out,
                 tok_ib, b0, b1, b2, b3, s0, s1, s2, s3, w0, w1, w2, w3):
    wid = lax.axis_index("s") * 2 + lax.axis_index("c")
    pltpu.sync_copy(tok_idx.at[wid], tok_ib)
    _gather_task(wid, word, tok_ib, tok_out, (b0, b1, b2, b3),
                 (s0, s1, s2, s3), (w0, w1, w2, w3),
                 TOK_NC, TOK_NC * K)


def _sc_et_body(et_idx, word, et_out,
                et_ib, b0, b1, b2, b3, acc, s0, s1, s2, s3):
    wid = lax.axis_index("s") * 2 + lax.axis_index("c")
    pltpu.sync_copy(et_idx.at[wid], et_ib)
    bufs = (b0, b1, b2, b3)
    sgs = (s0, s1, s2, s3)
    seg_chunks = ET_NC // ET_SEG
    seg_rows = seg_chunks * 8

    def g_start(c, i):
        pltpu.make_async_copy(word.at[et_ib.at[pl.ds(c * K, K)]],
                              bufs[i], sgs[i]).start()

    def g_wait(c, i):
        pltpu.make_async_copy(word.at[et_ib.at[pl.ds(c * K, K)]],
                              bufs[i], sgs[i]).wait()

    def reduce(buf, brow):
        def rbody(r, carry):
            for cc in range(8):
                sl = pl.ds(cc * 16, 16)
                v = buf[10 * r, sl]
                for m in range(1, 10):
                    v = v + buf[10 * r + m, sl]
                acc[brow + r, sl] = v
            return carry
        lax.fori_loop(0, 8, rbody, 0)

    def seg_body(seg, carry):
        ch0 = seg * seg_chunks
        g_start(ch0 + 0, 0)
        g_start(ch0 + 1, 1)
        g_start(ch0 + 2, 2)

        def quad(q, inner):
            for p in range(4):
                lc = 4 * q + p
                g_wait(ch0 + lc, p)
                reduce(bufs[p], lc * 8)

                @pl.when(lc + 3 < seg_chunks)
                def _(lc=lc, p=p):
                    g_start(ch0 + lc + 3, (p + 3) % 4)
            return inner

        lax.fori_loop(0, seg_chunks // 4, quad, 0)
        pltpu.sync_copy(acc,
                        et_out.at[pl.ds(wid * (ET_NC * 8) + seg * seg_rows,
                                        seg_rows)])
        return carry

    lax.fori_loop(0, ET_SEG, seg_body, 0)


def _sc_cd_body(cd_idx, ee_idx, ent, cd_out, ee_out,
                cd_ib, ee_ib, b0, b1, b2, b3,
                s0, s1, s2, s3, w0, w1, w2, w3):
    wid = lax.axis_index("s") * 2 + lax.axis_index("c")
    pltpu.sync_copy(cd_idx.at[wid], cd_ib)
    pltpu.sync_copy(ee_idx.at[wid], ee_ib)
    bufs = (b0, b1, b2, b3)
    sgs = (s0, s1, s2, s3)
    sws = (w0, w1, w2, w3)
    _gather_task(wid, ent, cd_ib, cd_out, bufs, sgs, sws, CD_NC, CD_NC * K)
    _gather_task(wid, ent, ee_ib, ee_out, bufs, sgs, sws, EE_NC, EE_NC * K)


_MESH = dict(core_axis_name="c", subcore_axis_name="s")


def _sems(n):
    return [pltpu.SemaphoreType.DMA] * n


def _bufs(n):
    return [pltpu.VMEM((K, H), jnp.float32)] * n


def _sc_tok(tok_idx, word_emb, bt):
    return pl.kernel(
        _sc_tok_body,
        out_type=jax.ShapeDtypeStruct((bt, H), jnp.float32),
        mesh=plsc.VectorSubcoreMesh(**_MESH),
        scratch_types=[pltpu.VMEM((TOK_NC * K,), jnp.int32)]
        + _bufs(4) + _sems(8),
    )(tok_idx, word_emb)


def _sc_et(et_idx, word_emb, be):
    return pl.kernel(
        _sc_et_body,
        out_type=jax.ShapeDtypeStruct((be, H), jnp.float32),
        mesh=plsc.VectorSubcoreMesh(**_MESH),
        scratch_types=[pltpu.VMEM((ET_NC * K,), jnp.int32)]
        + _bufs(4)
        + [pltpu.VMEM((ET_NC // ET_SEG * 8, H), jnp.float32)] + _sems(4),
    )(et_idx, word_emb)


def _sc_cd(cd_idx, ee_idx, ent_emb, bc, be):
    return pl.kernel(
        _sc_cd_body,
        out_type=[jax.ShapeDtypeStruct((bc, H), jnp.float32),
                  jax.ShapeDtypeStruct((be, H), jnp.float32)],
        mesh=plsc.VectorSubcoreMesh(**_MESH),
        scratch_types=[pltpu.VMEM((CD_NC * K,), jnp.int32),
                       pltpu.VMEM((1664,), jnp.int32)]
        + _bufs(4) + _sems(8),
    )(cd_idx, ee_idx, ent_emb)


def _ln(x, g, b):
    m = jnp.mean(x, axis=-1, keepdims=True)
    v = jnp.mean((x - m) ** 2, axis=-1, keepdims=True)
    return (x - m) / jnp.sqrt(v + EPS) * g + b


_DN = (((0,), (0,)), ((), ()))
_DNR = (((1,), (0,)), ((), ()))


def _onehot(idx_1xn, width, blk, dtype):
    iota = lax.broadcasted_iota(jnp.int32, (width, blk), 0)
    return (iota == idx_1xn).astype(dtype)


def _onehot_rows(idx_1xn, table_ref, width, blk):
    oh = _onehot(idx_1xn, width, blk, jnp.bfloat16)
    return lax.dot_general(oh, table_ref[...], _DN,
                           preferred_element_type=jnp.float32)


def _tok_tc_body(rows_ref, ints_ref, pos_ref, typ_ref, g_ref, b_ref,
                 out_ref, *, blk):
    x = rows_ref[...]
    x = x + _onehot_rows(ints_ref[0, 0:1, :], pos_ref, 256, blk)
    x = x + _onehot_rows(ints_ref[0, 1:2, :], typ_ref, 16, blk)
    out_ref[...] = _ln(x, g_ref[...], b_ref[...])


def _ent_tc_body(ee_ref, es_ref, ints_ref, mask_ref, w1_ref, w2_ref,
                 fb_ref, typ_ref, recip_ref, g_ref, b_ref,
                 out_ref, *, blk):
    g = g_ref[...]
    b = b_ref[...]
    midx = ints_ref[0, 0:1, :]
    tidx = ints_ref[0, 1:2, :]
    lidx = ints_ref[0, 2:3, :]
    oh_len = _onehot(lidx, 16, blk, jnp.float32)
    rlen = lax.dot_general(oh_len, recip_ref[...], _DN,
                           preferred_element_type=jnp.float32)
    oh_mask = _onehot(midx, 8, blk, jnp.float32)
    e0 = (lax.broadcasted_iota(jnp.int32, (8, 1), 0) == 0).astype(jnp.float32)
    mnz0 = lax.dot_general(oh_mask, e0, _DN,
                           preferred_element_type=jnp.float32)
    mrows = _onehot_rows(midx, mask_ref, 8, blk)
    et = mnz0 * (es_ref[...] * rlen) + (1.0 - mnz0) * mrows
    x = lax.dot_general(ee_ref[...].astype(jnp.bfloat16), w1_ref[...], _DNR,
                        preferred_element_type=jnp.float32)
    x = x + lax.dot_general(et.astype(jnp.bfloat16), w2_ref[...], _DNR,
                            preferred_element_type=jnp.float32)
    x = x + fb_ref[...]
    x = 0.5 * x * (1.0 + lax.erf(x * (2.0 ** -0.5)))
    x = _ln(x, g, b)
    x = x + _onehot_rows(tidx, typ_ref, 16, blk)
    out_ref[...] = _ln(x, g, b)


def kernel(input_tok, input_tok_type, input_tok_pos, input_ent_tok,
           input_ent_tok_length, input_ent_mask_type, input_ent,
           input_ent_type, ent_candidates, word_emb, ent_emb, pos_emb,
           type_emb, mask_emb, fusion_w, fusion_b, ln_g, ln_b):
    B, T = input_tok.shape
    _, E, M = input_ent_tok.shape
    _, C = ent_candidates.shape
    BT, BE, BC = B * T, B * E, B * C
    f32 = jnp.float32
    bf16 = jnp.bfloat16

    tok_idx = input_tok.reshape(NW, TOK_NC * K)
    et_idx = input_ent_tok.reshape(NW, ET_NC * K)
    ee_idx = jnp.pad(input_ent.reshape(NW, EE_NC * K), ((0, 0), (0, 64)))
    cd_idx = ent_candidates.reshape(NW, CD_NC * K)

    tok_rows = _sc_tok(tok_idx, word_emb, BT)
    et_sum = _sc_et(et_idx, word_emb, BE)
    cd_rows, ee_rows = _sc_cd(cd_idx, ee_idx, ent_emb, BC, BE)

    BLK = 512
    nb = BT // BLK
    pos256 = pos_emb[:256].astype(bf16)
    typ16 = jnp.zeros((16, H), f32).at[:type_emb.shape[0]].set(type_emb)
    typ16 = typ16.astype(bf16)
    ints1 = jnp.stack([input_tok_pos.reshape(nb, BLK),
                       input_tok_type.reshape(nb, BLK)], axis=1)
    ints1 = jnp.pad(ints1, ((0, 0), (0, 6), (0, 0)))
    g2 = ln_g.reshape(1, H)
    b2 = ln_b.reshape(1, H)

    tok_out = pl.pallas_call(
        functools.partial(_tok_tc_body, blk=BLK),
        grid=(nb,),
        in_specs=[
            pl.BlockSpec((BLK, H), lambda i: (i, 0)),
            pl.BlockSpec((1, 8, BLK), lambda i: (i, 0, 0)),
            pl.BlockSpec((256, H), lambda i: (0, 0)),
            pl.BlockSpec((16, H), lambda i: (0, 0)),
            pl.BlockSpec((1, H), lambda i: (0, 0)),
            pl.BlockSpec((1, H), lambda i: (0, 0)),
        ],
        out_specs=pl.BlockSpec((BLK, H), lambda i: (i, 0)),
        out_shape=jax.ShapeDtypeStruct((BT, H), f32),
    )(tok_rows, ints1, pos256, typ16, g2, b2)

    BLK2 = 512
    nb2 = BE // BLK2
    ints2 = jnp.stack([input_ent_mask_type.reshape(nb2, BLK2),
                       input_ent_type.reshape(nb2, BLK2),
                       input_ent_tok_length.reshape(nb2, BLK2)], axis=1)
    ints2 = jnp.pad(ints2, ((0, 0), (0, 5), (0, 0)))
    mask8 = jnp.zeros((8, H), f32).at[:mask_emb.shape[0]].set(mask_emb)
    mask8 = mask8.astype(bf16)
    w1 = fusion_w[:H].astype(bf16)
    w2 = fusion_w[H:].astype(bf16)
    fb2 = fusion_b.reshape(1, H)
    recip16 = (1.0 / jnp.maximum(jnp.arange(16, dtype=f32), 1.0)
               ).reshape(16, 1)

    ent_out = pl.pallas_call(
        functools.partial(_ent_tc_body, blk=BLK2),
        grid=(nb2,),
        in_specs=[
            pl.BlockSpec((BLK2, H), lambda i: (i, 0)),
            pl.BlockSpec((BLK2, H), lambda i: (i, 0)),
            pl.BlockSpec((1, 8, BLK2), lambda i: (i, 0, 0)),
            pl.BlockSpec((8, H), lambda i: (0, 0)),
            pl.BlockSpec((H, H), lambda i: (0, 0)),
            pl.BlockSpec((H, H), lambda i: (0, 0)),
            pl.BlockSpec((1, H), lambda i: (0, 0)),
            pl.BlockSpec((16, H), lambda i: (0, 0)),
            pl.BlockSpec((16, 1), lambda i: (0, 0)),
            pl.BlockSpec((1, H), lambda i: (0, 0)),
            pl.BlockSpec((1, H), lambda i: (0, 0)),
        ],
        out_specs=pl.BlockSpec((BLK2, H), lambda i: (i, 0)),
        out_shape=jax.ShapeDtypeStruct((BE, H), f32),
    )(ee_rows, et_sum, ints2, mask8, w1, w2, fb2, typ16, recip16, g2, b2)

    return (tok_out.reshape(B, T, H),
            ent_out.reshape(B, E, H),
            cd_rows.reshape(B, C, H))


# speedup vs baseline: 1.5471x; 1.1600x over previous
import functools

import jax
import jax.numpy as jnp
from jax import lax
from jax.experimental import pallas as pl
from jax.experimental.pallas import tpu as pltpu
from jax.experimental.pallas import tpu_sc as plsc

EPS = 1e-12
NW = 32
H = 128

K = 80
TOK_NC = 80
ET_NC = 200
ET_SEG = 5
EE_NC = 20
CD_NC = 40


def _gather_task(wid, table, idx_buf, out, bufs, sgs, sws, nchunks, rpw):
    base = wid * rpw

    def g_start(c, i):
        pltpu.make_async_copy(table.at[idx_buf.at[pl.ds(c * K, K)]],
                              bufs[i], sgs[i]).start()

    def g_wait(c, i):
        pltpu.make_async_copy(table.at[idx_buf.at[pl.ds(c * K, K)]],
                              bufs[i], sgs[i]).wait()

    def w_start(c, i):
        pltpu.make_async_copy(bufs[i], out.at[pl.ds(base + c * K, K)],
                              sws[i]).start()

    def w_wait(c, i):
        pltpu.make_async_copy(bufs[i], out.at[pl.ds(base + c * K, K)],
                              sws[i]).wait()

    for i in range(4):
        g_start(i, i)

    def penta(q, carry):
        for p in range(5):
            c = 5 * q + p
            pn = (p + 4) % 5
            g_wait(c, p)
            w_start(c, p)

            @pl.when(c > 0)
            def _(c=c, pn=pn):
                w_wait(c - 1, pn)

            @pl.when(c + 4 < nchunks)
            def _(c=c, pn=pn):
                g_start(c + 4, pn)
        return carry

    lax.fori_loop(0, nchunks // 5, penta, 0)
    w_wait(nchunks - 1, 4)


def _sc_tok_body(tok_idx, word, tok_out,
                 tok_ib, b0, b1, b2, b3, b4, s0, s1, s2, s3, s4,
                 w0, w1, w2, w3, w4):
    wid = lax.axis_index("s") * 2 + lax.axis_index("c")
    pltpu.sync_copy(tok_idx.at[wid], tok_ib)
    _gather_task(wid, word, tok_ib, tok_out, (b0, b1, b2, b3, b4),
                 (s0, s1, s2, s3, s4), (w0, w1, w2, w3, w4),
                 TOK_NC, TOK_NC * K)


def _sc_et_body(et_idx, word, et_out,
                et_ib, b0, b1, b2, b3, b4, acc, s0, s1, s2, s3, s4):
    wid = lax.axis_index("s") * 2 + lax.axis_index("c")
    pltpu.sync_copy(et_idx.at[wid], et_ib)
    bufs = (b0, b1, b2, b3, b4)
    sgs = (s0, s1, s2, s3, s4)
    seg_chunks = ET_NC // ET_SEG
    seg_rows = seg_chunks * 8

    def g_start(c, i):
        pltpu.make_async_copy(word.at[et_ib.at[pl.ds(c * K, K)]],
                              bufs[i], sgs[i]).start()

    def g_wait(c, i):
        pltpu.make_async_copy(word.at[et_ib.at[pl.ds(c * K, K)]],
                              bufs[i], sgs[i]).wait()

    def reduce(buf, brow):
        def rbody(r, carry):
            for cc in range(8):
                sl = pl.ds(cc * 16, 16)
                v = buf[10 * r, sl]
                for m in range(1, 10):
                    v = v + buf[10 * r + m, sl]
                acc[brow + r, sl] = v
            return carry
        lax.fori_loop(0, 8, rbody, 0)

    def seg_body(seg, carry):
        ch0 = seg * seg_chunks
        for i in range(4):
            g_start(ch0 + i, i)

        def penta(q, inner):
            for p in range(5):
                lc = 5 * q + p
                g_wait(ch0 + lc, p)
                reduce(bufs[p], lc * 8)

                @pl.when(lc + 4 < seg_chunks)
                def _(lc=lc, p=p):
                    g_start(ch0 + lc + 4, (p + 4) % 5)
            return inner

        lax.fori_loop(0, seg_chunks // 5, penta, 0)
        pltpu.sync_copy(acc,
                        et_out.at[pl.ds(wid * (ET_NC * 8) + seg * seg_rows,
                                        seg_rows)])
        return carry

    lax.fori_loop(0, ET_SEG, seg_body, 0)


def _sc_cd_body(cd_idx, ee_idx, ent, cd_out, ee_out,
                cd_ib, ee_ib, b0, b1, b2, b3, b4,
                s0, s1, s2, s3, s4, w0, w1, w2, w3, w4):
    wid = lax.axis_index("s") * 2 + lax.axis_index("c")
    pltpu.sync_copy(cd_idx.at[wid], cd_ib)
    pltpu.sync_copy(ee_idx.at[wid], ee_ib)
    bufs = (b0, b1, b2, b3, b4)
    sgs = (s0, s1, s2, s3, s4)
    sws = (w0, w1, w2, w3, w4)
    _gather_task(wid, ent, cd_ib, cd_out, bufs, sgs, sws, CD_NC, CD_NC * K)
    _gather_task(wid, ent, ee_ib, ee_out, bufs, sgs, sws, EE_NC, EE_NC * K)


_MESH = dict(core_axis_name="c", subcore_axis_name="s")


def _sems(n):
    return [pltpu.SemaphoreType.DMA] * n


def _bufs(n):
    return [pltpu.VMEM((K, H), jnp.float32)] * n


def _sc_tok(tok_idx, word_emb, bt):
    return pl.kernel(
        _sc_tok_body,
        out_type=jax.ShapeDtypeStruct((bt, H), jnp.float32),
        mesh=plsc.VectorSubcoreMesh(**_MESH),
        scratch_types=[pltpu.VMEM((TOK_NC * K,), jnp.int32)]
        + _bufs(5) + _sems(10),
    )(tok_idx, word_emb)


def _sc_et(et_idx, word_emb, be):
    return pl.kernel(
        _sc_et_body,
        out_type=jax.ShapeDtypeStruct((be, H), jnp.float32),
        mesh=plsc.VectorSubcoreMesh(**_MESH),
        scratch_types=[pltpu.VMEM((ET_NC * K,), jnp.int32)]
        + _bufs(5)
        + [pltpu.VMEM((ET_NC // ET_SEG * 8, H), jnp.float32)] + _sems(5),
    )(et_idx, word_emb)


def _sc_cd(cd_idx, ee_idx, ent_emb, bc, be):
    return pl.kernel(
        _sc_cd_body,
        out_type=[jax.ShapeDtypeStruct((bc, H), jnp.float32),
                  jax.ShapeDtypeStruct((be, H), jnp.float32)],
        mesh=plsc.VectorSubcoreMesh(**_MESH),
        scratch_types=[pltpu.VMEM((CD_NC * K,), jnp.int32),
                       pltpu.VMEM((1664,), jnp.int32)]
        + _bufs(5) + _sems(10),
    )(cd_idx, ee_idx, ent_emb)


def _ln(x, g, b):
    m = jnp.mean(x, axis=-1, keepdims=True)
    v = jnp.mean((x - m) ** 2, axis=-1, keepdims=True)
    return (x - m) / jnp.sqrt(v + EPS) * g + b


_DN = (((0,), (0,)), ((), ()))
_DNR = (((1,), (0,)), ((), ()))


def _onehot(idx_1xn, width, blk, dtype):
    iota = lax.broadcasted_iota(jnp.int32, (width, blk), 0)
    return (iota == idx_1xn).astype(dtype)


def _onehot_rows(idx_1xn, table_ref, width, blk):
    oh = _onehot(idx_1xn, width, blk, jnp.bfloat16)
    return lax.dot_general(oh, table_ref[...], _DN,
                           preferred_element_type=jnp.float32)


def _tok_tc_body(rows_ref, ints_ref, pos_ref, typ_ref, g_ref, b_ref,
                 out_ref, *, blk):
    x = rows_ref[...]
    x = x + _onehot_rows(ints_ref[0, 0:1, :], pos_ref, 256, blk)
    x = x + _onehot_rows(ints_ref[0, 1:2, :], typ_ref, 16, blk)
    out_ref[...] = _ln(x, g_ref[...], b_ref[...])


def _ent_tc_body(ee_ref, es_ref, ints_ref, mask_ref, w1_ref, w2_ref,
                 fb_ref, typ_ref, recip_ref, g_ref, b_ref,
                 out_ref, *, blk):
    g = g_ref[...]
    b = b_ref[...]
    midx = ints_ref[0, 0:1, :]
    tidx = ints_ref[0, 1:2, :]
    lidx = ints_ref[0, 2:3, :]
    oh_len = _onehot(lidx, 16, blk, jnp.float32)
    rlen = lax.dot_general(oh_len, recip_ref[...], _DN,
                           preferred_element_type=jnp.float32)
    oh_mask = _onehot(midx, 8, blk, jnp.float32)
    e0 = (lax.broadcasted_iota(jnp.int32, (8, 1), 0) == 0).astype(jnp.float32)
    mnz0 = lax.dot_general(oh_mask, e0, _DN,
                           preferred_element_type=jnp.float32)
    mrows = _onehot_rows(midx, mask_ref, 8, blk)
    et = mnz0 * (es_ref[...] * rlen) + (1.0 - mnz0) * mrows
    x = lax.dot_general(ee_ref[...].astype(jnp.bfloat16), w1_ref[...], _DNR,
                        preferred_element_type=jnp.float32)
    x = x + lax.dot_general(et.astype(jnp.bfloat16), w2_ref[...], _DNR,
                            preferred_element_type=jnp.float32)
    x = x + fb_ref[...]
    x = 0.5 * x * (1.0 + lax.erf(x * (2.0 ** -0.5)))
    x = _ln(x, g, b)
    x = x + _onehot_rows(tidx, typ_ref, 16, blk)
    out_ref[...] = _ln(x, g, b)


def kernel(input_tok, input_tok_type, input_tok_pos, input_ent_tok,
           input_ent_tok_length, input_ent_mask_type, input_ent,
           input_ent_type, ent_candidates, word_emb, ent_emb, pos_emb,
           type_emb, mask_emb, fusion_w, fusion_b, ln_g, ln_b):
    B, T = input_tok.shape
    _, E, M = input_ent_tok.shape
    _, C = ent_candidates.shape
    BT, BE, BC = B * T, B * E, B * C
    f32 = jnp.float32
    bf16 = jnp.bfloat16

    tok_idx = input_tok.reshape(NW, TOK_NC * K)
    et_idx = input_ent_tok.reshape(NW, ET_NC * K)
    ee_idx = jnp.pad(input_ent.reshape(NW, EE_NC * K), ((0, 0), (0, 64)))
    cd_idx = ent_candidates.reshape(NW, CD_NC * K)

    tok_rows = _sc_tok(tok_idx, word_emb, BT)
    et_sum = _sc_et(et_idx, word_emb, BE)
    cd_rows, ee_rows = _sc_cd(cd_idx, ee_idx, ent_emb, BC, BE)

    BLK = 1024
    nb = BT // BLK
    pos256 = pos_emb[:256].astype(bf16)
    typ16 = jnp.zeros((16, H), f32).at[:type_emb.shape[0]].set(type_emb)
    typ16 = typ16.astype(bf16)
    ints1 = jnp.stack([input_tok_pos.reshape(nb, BLK),
                       input_tok_type.reshape(nb, BLK)], axis=1)
    ints1 = jnp.pad(ints1, ((0, 0), (0, 6), (0, 0)))
    g2 = ln_g.reshape(1, H)
    b2 = ln_b.reshape(1, H)

    tok_out = pl.pallas_call(
        functools.partial(_tok_tc_body, blk=BLK),
        grid=(nb,),
        in_specs=[
            pl.BlockSpec((BLK, H), lambda i: (i, 0)),
            pl.BlockSpec((1, 8, BLK), lambda i: (i, 0, 0)),
            pl.BlockSpec((256, H), lambda i: (0, 0)),
            pl.BlockSpec((16, H), lambda i: (0, 0)),
            pl.BlockSpec((1, H), lambda i: (0, 0)),
            pl.BlockSpec((1, H), lambda i: (0, 0)),
        ],
        out_specs=pl.BlockSpec((BLK, H), lambda i: (i, 0)),
        out_shape=jax.ShapeDtypeStruct((BT, H), f32),
    )(tok_rows, ints1, pos256, typ16, g2, b2)

    BLK2 = 1024
    nb2 = BE // BLK2
    ints2 = jnp.stack([input_ent_mask_type.reshape(nb2, BLK2),
                       input_ent_type.reshape(nb2, BLK2),
                       input_ent_tok_length.reshape(nb2, BLK2)], axis=1)
    ints2 = jnp.pad(ints2, ((0, 0), (0, 5), (0, 0)))
    mask8 = jnp.zeros((8, H), f32).at[:mask_emb.shape[0]].set(mask_emb)
    mask8 = mask8.astype(bf16)
    w1 = fusion_w[:H].astype(bf16)
    w2 = fusion_w[H:].astype(bf16)
    fb2 = fusion_b.reshape(1, H)
    recip16 = (1.0 / jnp.maximum(jnp.arange(16, dtype=f32), 1.0)
               ).reshape(16, 1)

    ent_out = pl.pallas_call(
        functools.partial(_ent_tc_body, blk=BLK2),
        grid=(nb2,),
        in_specs=[
            pl.BlockSpec((BLK2, H), lambda i: (i, 0)),
            pl.BlockSpec((BLK2, H), lambda i: (i, 0)),
            pl.BlockSpec((1, 8, BLK2), lambda i: (i, 0, 0)),
            pl.BlockSpec((8, H), lambda i: (0, 0)),
            pl.BlockSpec((H, H), lambda i: (0, 0)),
            pl.BlockSpec((H, H), lambda i: (0, 0)),
            pl.BlockSpec((1, H), lambda i: (0, 0)),
            pl.BlockSpec((16, H), lambda i: (0, 0)),
            pl.BlockSpec((16, 1), lambda i: (0, 0)),
            pl.BlockSpec((1, H), lambda i: (0, 0)),
            pl.BlockSpec((1, H), lambda i: (0, 0)),
        ],
        out_specs=pl.BlockSpec((BLK2, H), lambda i: (i, 0)),
        out_shape=jax.ShapeDtypeStruct((BE, H), f32),
    )(ee_rows, et_sum, ints2, mask8, w1, w2, fb2, typ16, recip16, g2, b2)

    return (tok_out.reshape(B, T, H),
            ent_out.reshape(B, E, H),
            cd_rows.reshape(B, C, H))


# speedup vs baseline: 1.5921x; 1.0291x over previous
import functools

import jax
import jax.numpy as jnp
from jax import lax
from jax.experimental import pallas as pl
from jax.experimental.pallas import tpu as pltpu
from jax.experimental.pallas import tpu_sc as plsc

EPS = 1e-12
NW = 32
H = 128

K = 80
TOK_NC = 80
ET_NC = 200
ET_SEG = 5
EE_NC = 20
CD_NC = 40


def _gather_task(wid, table, idx_buf, out, bufs, sgs, sws, nchunks, rpw):
    base = wid * rpw

    def g_start(c, i):
        pltpu.make_async_copy(table.at[idx_buf.at[pl.ds(c * K, K)]],
                              bufs[i], sgs[i]).start()

    def g_wait(c, i):
        pltpu.make_async_copy(table.at[idx_buf.at[pl.ds(c * K, K)]],
                              bufs[i], sgs[i]).wait()

    def w_start(c, i):
        pltpu.make_async_copy(bufs[i], out.at[pl.ds(base + c * K, K)],
                              sws[i]).start()

    def w_wait(c, i):
        pltpu.make_async_copy(bufs[i], out.at[pl.ds(base + c * K, K)],
                              sws[i]).wait()

    for i in range(3):
        g_start(i, i)

    def quad(q, carry):
        for p in range(4):
            c = 4 * q + p
            pn = (p + 3) % 4
            g_wait(c, p)
            w_start(c, p)

            @pl.when(c > 0)
            def _(c=c, pn=pn):
                w_wait(c - 1, pn)

            @pl.when(c + 3 < nchunks)
            def _(c=c, pn=pn):
                g_start(c + 3, pn)
        return carry

    lax.fori_loop(0, nchunks // 4, quad, 0)
    w_wait(nchunks - 1, 3)


def _sc_tok_body(tok_idx, word, tok_out,
                 tok_ib, b0, b1, b2, b3, s0, s1, s2, s3,
                 w0, w1, w2, w3):
    wid = lax.axis_index("s") * 2 + lax.axis_index("c")
    pltpu.sync_copy(tok_idx.at[wid], tok_ib)
    _gather_task(wid, word, tok_ib, tok_out, (b0, b1, b2, b3),
                 (s0, s1, s2, s3), (w0, w1, w2, w3),
                 TOK_NC, TOK_NC * K)


def _sc_et_body(et_idx, word, et_out,
                et_ib, b0, b1, b2, b3, acc, s0, s1, s2, s3):
    wid = lax.axis_index("s") * 2 + lax.axis_index("c")
    pltpu.sync_copy(et_idx.at[wid], et_ib)
    bufs = (b0, b1, b2, b3)
    sgs = (s0, s1, s2, s3)
    seg_chunks = ET_NC // ET_SEG
    seg_rows = seg_chunks * 8

    def g_start(c, i):
        pltpu.make_async_copy(word.at[et_ib.at[pl.ds(c * K, K)]],
                              bufs[i], sgs[i]).start()

    def g_wait(c, i):
        pltpu.make_async_copy(word.at[et_ib.at[pl.ds(c * K, K)]],
                              bufs[i], sgs[i]).wait()

    def reduce(buf, brow):
        def rbody(r, carry):
            for cc in range(8):
                sl = pl.ds(cc * 16, 16)
                v = buf[10 * r, sl]
                for m in range(1, 10):
                    v = v + buf[10 * r + m, sl]
                acc[brow + r, sl] = v
            return carry
        lax.fori_loop(0, 8, rbody, 0)

    def seg_body(seg, carry):
        ch0 = seg * seg_chunks
        for i in range(3):
            g_start(ch0 + i, i)

        def quad(q, inner):
            for p in range(4):
                lc = 4 * q + p
                g_wait(ch0 + lc, p)
                reduce(bufs[p], lc * 8)

                @pl.when(lc + 3 < seg_chunks)
                def _(lc=lc, p=p):
                    g_start(ch0 + lc + 3, (p + 3) % 4)
            return inner

        lax.fori_loop(0, seg_chunks // 4, quad, 0)
        pltpu.sync_copy(acc,
                        et_out.at[pl.ds(wid * (ET_NC * 8) + seg * seg_rows,
                                        seg_rows)])
        return carry

    lax.fori_loop(0, ET_SEG, seg_body, 0)


def _sc_cd_body(cd_idx, ee_idx, ent, cd_out, ee_out,
                cd_ib, ee_ib, b0, b1, b2, b3,
                s0, s1, s2, s3, w0, w1, w2, w3):
    wid = lax.axis_index("s") * 2 + lax.axis_index("c")
    pltpu.sync_copy(cd_idx.at[wid], cd_ib)
    pltpu.sync_copy(ee_idx.at[wid], ee_ib)
    bufs = (b0, b1, b2, b3)
    sgs = (s0, s1, s2, s3)
    sws = (w0, w1, w2, w3)
    _gather_task(wid, ent, cd_ib, cd_out, bufs, sgs, sws, CD_NC, CD_NC * K)
    _gather_task(wid, ent, ee_ib, ee_out, bufs, sgs, sws, EE_NC, EE_NC * K)


_MESH = dict(core_axis_name="c", subcore_axis_name="s")


def _sems(n):
    return [pltpu.SemaphoreType.DMA] * n


def _bufs(n):
    return [pltpu.VMEM((K, H), jnp.float32)] * n


def _sc_tok(tok_idx, word_emb, bt):
    return pl.kernel(
        _sc_tok_body,
        out_type=jax.ShapeDtypeStruct((bt, H), jnp.float32),
        mesh=plsc.VectorSubcoreMesh(**_MESH),
        scratch_types=[pltpu.VMEM((TOK_NC * K,), jnp.int32)]
        + _bufs(4) + _sems(8),
    )(tok_idx, word_emb)


def _sc_et(et_idx, word_emb, be):
    return pl.kernel(
        _sc_et_body,
        out_type=jax.ShapeDtypeStruct((be, H), jnp.float32),
        mesh=plsc.VectorSubcoreMesh(**_MESH),
        scratch_types=[pltpu.VMEM((ET_NC * K,), jnp.int32)]
        + _bufs(4)
        + [pltpu.VMEM((ET_NC // ET_SEG * 8, H), jnp.float32)] + _sems(4),
    )(et_idx, word_emb)


def _sc_cd(cd_idx, ee_idx, ent_emb, bc, be):
    return pl.kernel(
        _sc_cd_body,
        out_type=[jax.ShapeDtypeStruct((bc, H), jnp.float32),
                  jax.ShapeDtypeStruct((be, H), jnp.float32)],
        mesh=plsc.VectorSubcoreMesh(**_MESH),
        scratch_types=[pltpu.VMEM((CD_NC * K,), jnp.int32),
                       pltpu.VMEM((1664,), jnp.int32)]
        + _bufs(4) + _sems(8),
    )(cd_idx, ee_idx, ent_emb)


def _ln(x, g, b):
    m = jnp.mean(x, axis=-1, keepdims=True)
    v = jnp.mean((x - m) ** 2, axis=-1, keepdims=True)
    return (x - m) / jnp.sqrt(v + EPS) * g + b


_DN = (((0,), (0,)), ((), ()))
_DNR = (((1,), (0,)), ((), ()))


def _onehot(idx_1xn, width, blk, dtype):
    iota = lax.broadcasted_iota(jnp.int32, (width, blk), 0)
    return (iota == idx_1xn).astype(dtype)


def _onehot_rows(idx_1xn, table_ref, width, blk):
    oh = _onehot(idx_1xn, width, blk, jnp.bfloat16)
    return lax.dot_general(oh, table_ref[...], _DN,
                           preferred_element_type=jnp.float32)


def _tok_tc_body(rows_ref, ints_ref, pos_ref, typ_ref, g_ref, b_ref,
                 out_ref, *, blk):
    x = rows_ref[...]
    x = x + _onehot_rows(ints_ref[0, 0:1, :], pos_ref, 256, blk)
    x = x + _onehot_rows(ints_ref[0, 1:2, :], typ_ref, 16, blk)
    out_ref[...] = _ln(x, g_ref[...], b_ref[...])


def _ent_tc_body(ee_ref, es_ref, ints_ref, mask_ref, w1_ref, w2_ref,
                 fb_ref, typ_ref, recip_ref, g_ref, b_ref,
                 out_ref, *, blk):
    g = g_ref[...]
    b = b_ref[...]
    midx = ints_ref[0, 0:1, :]
    tidx = ints_ref[0, 1:2, :]
    lidx = ints_ref[0, 2:3, :]
    oh_len = _onehot(lidx, 16, blk, jnp.float32)
    rlen = lax.dot_general(oh_len, recip_ref[...], _DN,
                           preferred_element_type=jnp.float32)
    oh_mask = _onehot(midx, 8, blk, jnp.float32)
    e0 = (lax.broadcasted_iota(jnp.int32, (8, 1), 0) == 0).astype(jnp.float32)
    mnz0 = lax.dot_general(oh_mask, e0, _DN,
                           preferred_element_type=jnp.float32)
    mrows = _onehot_rows(midx, mask_ref, 8, blk)
    et = mnz0 * (es_ref[...] * rlen) + (1.0 - mnz0) * mrows
    x = lax.dot_general(ee_ref[...].astype(jnp.bfloat16), w1_ref[...], _DNR,
                        preferred_element_type=jnp.float32)
    x = x + lax.dot_general(et.astype(jnp.bfloat16), w2_ref[...], _DNR,
                            preferred_element_type=jnp.float32)
    x = x + fb_ref[...]
    x = 0.5 * x * (1.0 + lax.erf(x * (2.0 ** -0.5)))
    x = _ln(x, g, b)
    x = x + _onehot_rows(tidx, typ_ref, 16, blk)
    out_ref[...] = _ln(x, g, b)


def kernel(input_tok, input_tok_type, input_tok_pos, input_ent_tok,
           input_ent_tok_length, input_ent_mask_type, input_ent,
           input_ent_type, ent_candidates, word_emb, ent_emb, pos_emb,
           type_emb, mask_emb, fusion_w, fusion_b, ln_g, ln_b):
    B, T = input_tok.shape
    _, E, M = input_ent_tok.shape
    _, C = ent_candidates.shape
    BT, BE, BC = B * T, B * E, B * C
    f32 = jnp.float32
    bf16 = jnp.bfloat16

    tok_idx = input_tok.reshape(NW, TOK_NC * K)
    et_idx = input_ent_tok.reshape(NW, ET_NC * K)
    ee_idx = jnp.pad(input_ent.reshape(NW, EE_NC * K), ((0, 0), (0, 64)))
    cd_idx = ent_candidates.reshape(NW, CD_NC * K)

    tok_rows = _sc_tok(tok_idx, word_emb, BT)
    et_sum = _sc_et(et_idx, word_emb, BE)

    BLK = 1024
    nb = BT // BLK
    pos256 = pos_emb[:256].astype(bf16)
    typ16 = jnp.zeros((16, H), f32).at[:type_emb.shape[0]].set(type_emb)
    typ16 = typ16.astype(bf16)
    ints1 = jnp.stack([input_tok_pos.reshape(nb, BLK),
                       input_tok_type.reshape(nb, BLK)], axis=1)
    ints1 = jnp.pad(ints1, ((0, 0), (0, 6), (0, 0)))
    g2 = ln_g.reshape(1, H)
    b2 = ln_b.reshape(1, H)

    tok_out = pl.pallas_call(
        functools.partial(_tok_tc_body, blk=BLK),
        grid=(nb,),
        in_specs=[
            pl.BlockSpec((BLK, H), lambda i: (i, 0)),
            pl.BlockSpec((1, 8, BLK), lambda i: (i, 0, 0)),
            pl.BlockSpec((256, H), lambda i: (0, 0)),
            pl.BlockSpec((16, H), lambda i: (0, 0)),
            pl.BlockSpec((1, H), lambda i: (0, 0)),
            pl.BlockSpec((1, H), lambda i: (0, 0)),
        ],
        out_specs=pl.BlockSpec((BLK, H), lambda i: (i, 0)),
        out_shape=jax.ShapeDtypeStruct((BT, H), f32),
    )(tok_rows, ints1, pos256, typ16, g2, b2)

    cd_rows, ee_rows = _sc_cd(cd_idx, ee_idx, ent_emb, BC, BE)

    BLK2 = 1024
    nb2 = BE // BLK2
    ints2 = jnp.stack([input_ent_mask_type.reshape(nb2, BLK2),
                       input_ent_type.reshape(nb2, BLK2),
                       input_ent_tok_length.reshape(nb2, BLK2)], axis=1)
    ints2 = jnp.pad(ints2, ((0, 0), (0, 5), (0, 0)))
    mask8 = jnp.zeros((8, H), f32).at[:mask_emb.shape[0]].set(mask_emb)
    mask8 = mask8.astype(bf16)
    w1 = fusion_w[:H].astype(bf16)
    w2 = fusion_w[H:].astype(bf16)
    fb2 = fusion_b.reshape(1, H)
    recip16 = (1.0 / jnp.maximum(jnp.arange(16, dtype=f32), 1.0)
               ).reshape(16, 1)

    ent_out = pl.pallas_call(
        functools.partial(_ent_tc_body, blk=BLK2),
        grid=(nb2,),
        in_specs=[
            pl.BlockSpec((BLK2, H), lambda i: (i, 0)),
            pl.BlockSpec((BLK2, H), lambda i: (i, 0)),
            pl.BlockSpec((1, 8, BLK2), lambda i: (i, 0, 0)),
            pl.BlockSpec((8, H), lambda i: (0, 0)),
            pl.BlockSpec((H, H), lambda i: (0, 0)),
            pl.BlockSpec((H, H), lambda i: (0, 0)),
            pl.BlockSpec((1, H), lambda i: (0, 0)),
            pl.BlockSpec((16, H), lambda i: (0, 0)),
            pl.BlockSpec((16, 1), lambda i: (0, 0)),
            pl.BlockSpec((1, H), lambda i: (0, 0)),
            pl.BlockSpec((1, H), lambda i: (0, 0)),
        ],
        out_specs=pl.BlockSpec((BLK2, H), lambda i: (i, 0)),
        out_shape=jax.ShapeDtypeStruct((BE, H), f32),
    )(ee_rows, et_sum, ints2, mask8, w1, w2, fb2, typ16, recip16, g2, b2)

    return (tok_out.reshape(B, T, H),
            ent_out.reshape(B, E, H),
            cd_rows.reshape(B, C, H))


# speedup vs baseline: 1.7623x; 1.1069x over previous
import functools

import jax
import jax.numpy as jnp
from jax import lax
from jax.experimental import pallas as pl
from jax.experimental.pallas import tpu as pltpu
from jax.experimental.pallas import tpu_sc as plsc

EPS = 1e-12
NW = 32
H = 128

K = 80
TOK_NC = 80
ET_NC = 200
ET_SEG = 5
EE_NC = 20
CD_NC = 40


def _gather_task(wid, table, idx_buf, out, bufs, sgs, sws, nchunks, rpw):
    base = wid * rpw

    def g_start(c, i):
        pltpu.make_async_copy(table.at[idx_buf.at[pl.ds(c * K, K)]],
                              bufs[i], sgs[i]).start()

    def g_wait(c, i):
        pltpu.make_async_copy(table.at[idx_buf.at[pl.ds(c * K, K)]],
                              bufs[i], sgs[i]).wait()

    def w_start(c, i):
        pltpu.make_async_copy(bufs[i], out.at[pl.ds(base + c * K, K)],
                              sws[i]).start()

    def w_wait(c, i):
        pltpu.make_async_copy(bufs[i], out.at[pl.ds(base + c * K, K)],
                              sws[i]).wait()

    for i in range(3):
        g_start(i, i)

    def quad(q, carry):
        for p in range(4):
            c = 4 * q + p
            pn = (p + 3) % 4
            g_wait(c, p)
            w_start(c, p)

            @pl.when(c > 0)
            def _(c=c, pn=pn):
                w_wait(c - 1, pn)

            @pl.when(c + 3 < nchunks)
            def _(c=c, pn=pn):
                g_start(c + 3, pn)
        return carry

    lax.fori_loop(0, nchunks // 4, quad, 0)
    w_wait(nchunks - 1, 3)


def _sc_tok_body(tok_idx, word, tok_out,
                 tok_ib, b0, b1, b2, b3, s0, s1, s2, s3,
                 w0, w1, w2, w3):
    wid = lax.axis_index("s") * 2 + lax.axis_index("c")
    pltpu.sync_copy(tok_idx.at[wid], tok_ib)
    _gather_task(wid, word, tok_ib, tok_out, (b0, b1, b2, b3),
                 (s0, s1, s2, s3), (w0, w1, w2, w3),
                 TOK_NC, TOK_NC * K)


def _sc_et_body(et_idx, word, et_out,
                et_ib, b0, b1, b2, b3, acc, s0, s1, s2, s3):
    wid = lax.axis_index("s") * 2 + lax.axis_index("c")
    pltpu.sync_copy(et_idx.at[wid], et_ib)
    bufs = (b0, b1, b2, b3)
    sgs = (s0, s1, s2, s3)
    seg_chunks = ET_NC // ET_SEG
    seg_rows = seg_chunks * 8

    def g_start(c, i):
        pltpu.make_async_copy(word.at[et_ib.at[pl.ds(c * K, K)]],
                              bufs[i], sgs[i]).start()

    def g_wait(c, i):
        pltpu.make_async_copy(word.at[et_ib.at[pl.ds(c * K, K)]],
                              bufs[i], sgs[i]).wait()

    def reduce(buf, brow):
        def rbody(r, carry):
            for cc in range(8):
                sl = pl.ds(cc * 16, 16)
                v = buf[10 * r, sl]
                for m in range(1, 10):
                    v = v + buf[10 * r + m, sl]
                acc[brow + r, sl] = v
            return carry
        lax.fori_loop(0, 8, rbody, 0)

    def seg_body(seg, carry):
        ch0 = seg * seg_chunks
        for i in range(3):
            g_start(ch0 + i, i)

        def quad(q, inner):
            for p in range(4):
                lc = 4 * q + p
                g_wait(ch0 + lc, p)
                reduce(bufs[p], lc * 8)

                @pl.when(lc + 3 < seg_chunks)
                def _(lc=lc, p=p):
                    g_start(ch0 + lc + 3, (p + 3) % 4)
            return inner

        lax.fori_loop(0, seg_chunks // 4, quad, 0)
        pltpu.sync_copy(acc,
                        et_out.at[pl.ds(wid * (ET_NC * 8) + seg * seg_rows,
                                        seg_rows)])
        return carry

    lax.fori_loop(0, ET_SEG, seg_body, 0)


def _sc_cd_body(cd_idx, ee_idx, ent, cd_out, ee_out,
                cd_ib, ee_ib, b0, b1, b2, b3,
                s0, s1, s2, s3, w0, w1, w2, w3):
    wid = lax.axis_index("s") * 2 + lax.axis_index("c")
    pltpu.sync_copy(cd_idx.at[wid], cd_ib)
    pltpu.sync_copy(ee_idx.at[wid], ee_ib)
    bufs = (b0, b1, b2, b3)
    sgs = (s0, s1, s2, s3)
    sws = (w0, w1, w2, w3)
    _gather_task(wid, ent, cd_ib, cd_out, bufs, sgs, sws, CD_NC, CD_NC * K)
    _gather_task(wid, ent, ee_ib, ee_out, bufs, sgs, sws, EE_NC, EE_NC * K)


_MESH = dict(core_axis_name="c", subcore_axis_name="s")


def _sems(n):
    return [pltpu.SemaphoreType.DMA] * n


def _bufs(n):
    return [pltpu.VMEM((K, H), jnp.float32)] * n


def _sc_tok(tok_idx, word_emb, bt):
    return pl.kernel(
        _sc_tok_body,
        out_type=jax.ShapeDtypeStruct((bt, H), jnp.float32),
        mesh=plsc.VectorSubcoreMesh(**_MESH),
        scratch_types=[pltpu.VMEM((TOK_NC * K,), jnp.int32)]
        + _bufs(4) + _sems(8),
    )(tok_idx, word_emb)


def _sc_et(et_idx, word_emb, be):
    return pl.kernel(
        _sc_et_body,
        out_type=jax.ShapeDtypeStruct((be, H), jnp.float32),
        mesh=plsc.VectorSubcoreMesh(**_MESH),
        scratch_types=[pltpu.VMEM((ET_NC * K,), jnp.int32)]
        + _bufs(4)
        + [pltpu.VMEM((ET_NC // ET_SEG * 8, H), jnp.float32)] + _sems(4),
    )(et_idx, word_emb)


def _sc_cd(cd_idx, ee_idx, ent_emb, bc, be):
    return pl.kernel(
        _sc_cd_body,
        out_type=[jax.ShapeDtypeStruct((bc, H), jnp.float32),
                  jax.ShapeDtypeStruct((be, H), jnp.float32)],
        mesh=plsc.VectorSubcoreMesh(**_MESH),
        scratch_types=[pltpu.VMEM((CD_NC * K,), jnp.int32),
                       pltpu.VMEM((1664,), jnp.int32)]
        + _bufs(4) + _sems(8),
    )(cd_idx, ee_idx, ent_emb)


def _ln(x, g, b):
    m = jnp.mean(x, axis=-1, keepdims=True)
    v = jnp.mean((x - m) ** 2, axis=-1, keepdims=True)
    return (x - m) / jnp.sqrt(v + EPS) * g + b


_DN = (((0,), (0,)), ((), ()))
_DNR = (((1,), (0,)), ((), ()))


def _onehot(idx_1xn, width, blk, dtype):
    iota = lax.broadcasted_iota(jnp.int32, (width, blk), 0)
    return (iota == idx_1xn).astype(dtype)


def _onehot_rows(idx_1xn, table_ref, width, blk):
    oh = _onehot(idx_1xn, width, blk, jnp.bfloat16)
    return lax.dot_general(oh, table_ref[...], _DN,
                           preferred_element_type=jnp.float32)


def _tok_tc_body(rows_ref, ints_ref, pos_ref, typ_ref, g_ref, b_ref,
                 out_ref, *, blk):
    x = rows_ref[...]
    x = x + _onehot_rows(ints_ref[0, 0:1, :], pos_ref, 256, blk)
    x = x + _onehot_rows(ints_ref[0, 1:2, :], typ_ref, 16, blk)
    out_ref[...] = _ln(x, g_ref[...], b_ref[...])


def _ent_tc_body(ee_ref, es_ref, ints_ref, mask_ref, w1_ref, w2_ref,
                 fb_ref, typ_ref, recip_ref, g_ref, b_ref,
                 out_ref, *, blk):
    g = g_ref[...]
    b = b_ref[...]
    midx = ints_ref[0, 0:1, :]
    tidx = ints_ref[0, 1:2, :]
    lidx = ints_ref[0, 2:3, :]
    oh_len = _onehot(lidx, 16, blk, jnp.float32)
    rlen = lax.dot_general(oh_len, recip_ref[...], _DN,
                           preferred_element_type=jnp.float32)
    oh_mask = _onehot(midx, 8, blk, jnp.float32)
    e0 = (lax.broadcasted_iota(jnp.int32, (8, 1), 0) == 0).astype(jnp.float32)
    mnz0 = lax.dot_general(oh_mask, e0, _DN,
                           preferred_element_type=jnp.float32)
    mrows = _onehot_rows(midx, mask_ref, 8, blk)
    et = mnz0 * (es_ref[...] * rlen) + (1.0 - mnz0) * mrows
    x = lax.dot_general(ee_ref[...].astype(jnp.bfloat16), w1_ref[...], _DNR,
                        preferred_element_type=jnp.float32)
    x = x + lax.dot_general(et.astype(jnp.bfloat16), w2_ref[...], _DNR,
                            preferred_element_type=jnp.float32)
    x = x + fb_ref[...]
    x = 0.5 * x * (1.0 + lax.erf(x * (2.0 ** -0.5)))
    x = _ln(x, g, b)
    x = x + _onehot_rows(tidx, typ_ref, 16, blk)
    out_ref[...] = _ln(x, g, b)


def kernel(input_tok, input_tok_type, input_tok_pos, input_ent_tok,
           input_ent_tok_length, input_ent_mask_type, input_ent,
           input_ent_type, ent_candidates, word_emb, ent_emb, pos_emb,
           type_emb, mask_emb, fusion_w, fusion_b, ln_g, ln_b):
    B, T = input_tok.shape
    _, E, M = input_ent_tok.shape
    _, C = ent_candidates.shape
    BT, BE, BC = B * T, B * E, B * C
    f32 = jnp.float32
    bf16 = jnp.bfloat16

    tok_idx = input_tok.reshape(NW, TOK_NC * K)
    et_idx = input_ent_tok.reshape(NW, ET_NC * K)
    ee_idx = jnp.pad(input_ent.reshape(NW, EE_NC * K), ((0, 0), (0, 64)))
    cd_idx = ent_candidates.reshape(NW, CD_NC * K)

    tok_rows = _sc_tok(tok_idx, word_emb, BT)
    et_sum = _sc_et(et_idx, word_emb, BE)

    BLK = 2048
    nb = BT // BLK
    pos256 = pos_emb[:256].astype(bf16)
    typ16 = jnp.zeros((16, H), f32).at[:type_emb.shape[0]].set(type_emb)
    typ16 = typ16.astype(bf16)
    ints1 = jnp.stack([input_tok_pos.reshape(nb, BLK),
                       input_tok_type.reshape(nb, BLK)], axis=1)
    ints1 = jnp.pad(ints1, ((0, 0), (0, 6), (0, 0)))
    g2 = ln_g.reshape(1, H)
    b2 = ln_b.reshape(1, H)

    tok_out = pl.pallas_call(
        functools.partial(_tok_tc_body, blk=BLK),
        grid=(nb,),
        in_specs=[
            pl.BlockSpec((BLK, H), lambda i: (i, 0)),
            pl.BlockSpec((1, 8, BLK), lambda i: (i, 0, 0)),
            pl.BlockSpec((256, H), lambda i: (0, 0)),
            pl.BlockSpec((16, H), lambda i: (0, 0)),
            pl.BlockSpec((1, H), lambda i: (0, 0)),
            pl.BlockSpec((1, H), lambda i: (0, 0)),
        ],
        out_specs=pl.BlockSpec((BLK, H), lambda i: (i, 0)),
        out_shape=jax.ShapeDtypeStruct((BT, H), f32),
    )(tok_rows, ints1, pos256, typ16, g2, b2)

    cd_rows, ee_rows = _sc_cd(cd_idx, ee_idx, ent_emb, BC, BE)

    BLK2 = 2048
    nb2 = BE // BLK2
    ints2 = jnp.stack([input_ent_mask_type.reshape(nb2, BLK2),
                       input_ent_type.reshape(nb2, BLK2),
                       input_ent_tok_length.reshape(nb2, BLK2)], axis=1)
    ints2 = jnp.pad(ints2, ((0, 0), (0, 5), (0, 0)))
    mask8 = jnp.zeros((8, H), f32).at[:mask_emb.shape[0]].set(mask_emb)
    mask8 = mask8.astype(bf16)
    w1 = fusion_w[:H].astype(bf16)
    w2 = fusion_w[H:].astype(bf16)
    fb2 = fusion_b.reshape(1, H)
    recip16 = (1.0 / jnp.maximum(jnp.arange(16, dtype=f32), 1.0)
               ).reshape(16, 1)

    ent_out = pl.pallas_call(
        functools.partial(_ent_tc_body, blk=BLK2),
        grid=(nb2,),
        in_specs=[
            pl.BlockSpec((BLK2, H), lambda i: (i, 0)),
            pl.BlockSpec((BLK2, H), lambda i: (i, 0)),
            pl.BlockSpec((1, 8, BLK2), lambda i: (i, 0, 0)),
            pl.BlockSpec((8, H), lambda i: (0, 0)),
            pl.BlockSpec((H, H), lambda i: (0, 0)),
            pl.BlockSpec((H, H), lambda i: (0, 0)),
            pl.BlockSpec((1, H), lambda i: (0, 0)),
            pl.BlockSpec((16, H), lambda i: (0, 0)),
            pl.BlockSpec((16, 1), lambda i: (0, 0)),
            pl.BlockSpec((1, H), lambda i: (0, 0)),
            pl.BlockSpec((1, H), lambda i: (0, 0)),
        ],
        out_specs=pl.BlockSpec((BLK2, H), lambda i: (i, 0)),
        out_shape=jax.ShapeDtypeStruct((BE, H), f32),
    )(ee_rows, et_sum, ints2, mask8, w1, w2, fb2, typ16, recip16, g2, b2)

    return (tok_out.reshape(B, T, H),
            ent_out.reshape(B, E, H),
            cd_rows.reshape(B, C, H))


# speedup vs baseline: 1.8470x; 1.0481x over previous
import functools

import jax
import jax.numpy as jnp
from jax import lax
from jax.experimental import pallas as pl
from jax.experimental.pallas import tpu as pltpu
from jax.experimental.pallas import tpu_sc as plsc

EPS = 1e-12
NW = 32
H = 128

K = 80
TOK_NC = 80
ET_NC = 200
ET_SEG = 5
EE_NC = 20
CD_NC = 40


def _gather_task(wid, table, idx_buf, out, bufs, sgs, sws, nchunks, rpw):
    base = wid * rpw

    def g_start(c, i):
        pltpu.make_async_copy(table.at[idx_buf.at[pl.ds(c * K, K)]],
                              bufs[i], sgs[i]).start()

    def g_wait(c, i):
        pltpu.make_async_copy(table.at[idx_buf.at[pl.ds(c * K, K)]],
                              bufs[i], sgs[i]).wait()

    def w_start(c, i):
        pltpu.make_async_copy(bufs[i], out.at[pl.ds(base + c * K, K)],
                              sws[i]).start()

    def w_wait(c, i):
        pltpu.make_async_copy(bufs[i], out.at[pl.ds(base + c * K, K)],
                              sws[i]).wait()

    for i in range(3):
        g_start(i, i)

    def quad(q, carry):
        for p in range(4):
            c = 4 * q + p
            pn = (p + 3) % 4
            g_wait(c, p)
            w_start(c, p)

            @pl.when(c > 0)
            def _(c=c, pn=pn):
                w_wait(c - 1, pn)

            @pl.when(c + 3 < nchunks)
            def _(c=c, pn=pn):
                g_start(c + 3, pn)
        return carry

    lax.fori_loop(0, nchunks // 4, quad, 0)
    w_wait(nchunks - 1, 3)


def _sc_tok_body(tok_idx, word, tok_out,
                 tok_ib, b0, b1, b2, b3, s0, s1, s2, s3,
                 w0, w1, w2, w3):
    wid = lax.axis_index("s") * 2 + lax.axis_index("c")
    pltpu.sync_copy(tok_idx.at[wid], tok_ib)
    _gather_task(wid, word, tok_ib, tok_out, (b0, b1, b2, b3),
                 (s0, s1, s2, s3), (w0, w1, w2, w3),
                 TOK_NC, TOK_NC * K)


def _sc_et_body(et_idx, word, et_out,
                et_ib, b0, b1, b2, b3, acc, s0, s1, s2, s3):
    wid = lax.axis_index("s") * 2 + lax.axis_index("c")
    pltpu.sync_copy(et_idx.at[wid], et_ib)
    bufs = (b0, b1, b2, b3)
    sgs = (s0, s1, s2, s3)
    seg_chunks = ET_NC // ET_SEG
    seg_rows = seg_chunks * 8

    def g_start(c, i):
        pltpu.make_async_copy(word.at[et_ib.at[pl.ds(c * K, K)]],
                              bufs[i], sgs[i]).start()

    def g_wait(c, i):
        pltpu.make_async_copy(word.at[et_ib.at[pl.ds(c * K, K)]],
                              bufs[i], sgs[i]).wait()

    def reduce(buf, brow):
        def rbody(r, carry):
            for cc in range(8):
                sl = pl.ds(cc * 16, 16)
                v = buf[10 * r, sl]
                for m in range(1, 10):
                    v = v + buf[10 * r + m, sl]
                acc[brow + r, sl] = v
            return carry
        lax.fori_loop(0, 8, rbody, 0)

    def seg_body(seg, carry):
        ch0 = seg * seg_chunks
        for i in range(3):
            g_start(ch0 + i, i)

        def quad(q, inner):
            for p in range(4):
                lc = 4 * q + p
                g_wait(ch0 + lc, p)
                reduce(bufs[p], lc * 8)

                @pl.when(lc + 3 < seg_chunks)
                def _(lc=lc, p=p):
                    g_start(ch0 + lc + 3, (p + 3) % 4)
            return inner

        lax.fori_loop(0, seg_chunks // 4, quad, 0)
        pltpu.sync_copy(acc,
                        et_out.at[pl.ds(wid * (ET_NC * 8) + seg * seg_rows,
                                        seg_rows)])
        return carry

    lax.fori_loop(0, ET_SEG, seg_body, 0)


def _sc_cd_body(cd_idx, ee_idx, ent, cd_out, ee_out,
                cd_ib, ee_ib, b0, b1, b2, b3,
                s0, s1, s2, s3, w0, w1, w2, w3):
    wid = lax.axis_index("s") * 2 + lax.axis_index("c")
    pltpu.sync_copy(cd_idx.at[wid], cd_ib)
    pltpu.sync_copy(ee_idx.at[wid], ee_ib)
    bufs = (b0, b1, b2, b3)
    sgs = (s0, s1, s2, s3)
    sws = (w0, w1, w2, w3)
    _gather_task(wid, ent, cd_ib, cd_out, bufs, sgs, sws, CD_NC, CD_NC * K)
    _gather_task(wid, ent, ee_ib, ee_out, bufs, sgs, sws, EE_NC, EE_NC * K)


_MESH = dict(core_axis_name="c", subcore_axis_name="s")


def _sems(n):
    return [pltpu.SemaphoreType.DMA] * n


def _bufs(n):
    return [pltpu.VMEM((K, H), jnp.float32)] * n


def _sc_tok(tok_idx, word_emb, bt):
    return pl.kernel(
        _sc_tok_body,
        out_type=jax.ShapeDtypeStruct((bt, H), jnp.float32),
        mesh=plsc.VectorSubcoreMesh(**_MESH),
        scratch_types=[pltpu.VMEM((TOK_NC * K,), jnp.int32)]
        + _bufs(4) + _sems(8),
    )(tok_idx, word_emb)


def _sc_et(et_idx, word_emb, be):
    return pl.kernel(
        _sc_et_body,
        out_type=jax.ShapeDtypeStruct((be, H), jnp.float32),
        mesh=plsc.VectorSubcoreMesh(**_MESH),
        scratch_types=[pltpu.VMEM((ET_NC * K,), jnp.int32)]
        + _bufs(4)
        + [pltpu.VMEM((ET_NC // ET_SEG * 8, H), jnp.float32)] + _sems(4),
    )(et_idx, word_emb)


def _sc_cd(cd_idx, ee_idx, ent_emb, bc, be):
    return pl.kernel(
        _sc_cd_body,
        out_type=[jax.ShapeDtypeStruct((bc, H), jnp.float32),
                  jax.ShapeDtypeStruct((be, H), jnp.float32)],
        mesh=plsc.VectorSubcoreMesh(**_MESH),
        scratch_types=[pltpu.VMEM((CD_NC * K,), jnp.int32),
                       pltpu.VMEM((1664,), jnp.int32)]
        + _bufs(4) + _sems(8),
    )(cd_idx, ee_idx, ent_emb)


def _ln(x, g, b):
    m = jnp.mean(x, axis=-1, keepdims=True)
    v = jnp.mean((x - m) ** 2, axis=-1, keepdims=True)
    return (x - m) / jnp.sqrt(v + EPS) * g + b


_DN = (((0,), (0,)), ((), ()))
_DNR = (((1,), (0,)), ((), ()))


def _onehot(idx_1xn, width, blk, dtype):
    iota = lax.broadcasted_iota(jnp.int32, (width, blk), 0)
    return (iota == idx_1xn).astype(dtype)


def _onehot_rows(idx_1xn, table_ref, width, blk):
    oh = _onehot(idx_1xn, width, blk, jnp.bfloat16)
    return lax.dot_general(oh, table_ref[...], _DN,
                           preferred_element_type=jnp.float32)


def _tok_tc_body(rows_ref, ints_ref, pos_ref, typ_ref, g_ref, b_ref,
                 out_ref, *, blk):
    x = rows_ref[...]
    x = x + _onehot_rows(ints_ref[0, 0:1, :], pos_ref, 256, blk)
    x = x + _onehot_rows(ints_ref[0, 1:2, :], typ_ref, 16, blk)
    out_ref[...] = _ln(x, g_ref[...], b_ref[...])


def _ent_tc_body(ee_ref, es_ref, ints_ref, mask_ref, w1_ref, w2_ref,
                 fb_ref, typ_ref, recip_ref, g_ref, b_ref,
                 out_ref, *, blk):
    g = g_ref[...]
    b = b_ref[...]
    midx = ints_ref[0, 0:1, :]
    tidx = ints_ref[0, 1:2, :]
    lidx = ints_ref[0, 2:3, :]
    oh_len = _onehot(lidx, 16, blk, jnp.float32)
    rlen = lax.dot_general(oh_len, recip_ref[...], _DN,
                           preferred_element_type=jnp.float32)
    oh_mask = _onehot(midx, 8, blk, jnp.float32)
    e0 = (lax.broadcasted_iota(jnp.int32, (8, 1), 0) == 0).astype(jnp.float32)
    mnz0 = lax.dot_general(oh_mask, e0, _DN,
                           preferred_element_type=jnp.float32)
    mrows = _onehot_rows(midx, mask_ref, 8, blk)
    et = mnz0 * (es_ref[...] * rlen) + (1.0 - mnz0) * mrows
    x = lax.dot_general(ee_ref[...].astype(jnp.bfloat16), w1_ref[...], _DNR,
                        preferred_element_type=jnp.float32)
    x = x + lax.dot_general(et.astype(jnp.bfloat16), w2_ref[...], _DNR,
                            preferred_element_type=jnp.float32)
    x = x + fb_ref[...]
    x = 0.5 * x * (1.0 + lax.erf(x * (2.0 ** -0.5)))
    x = _ln(x, g, b)
    x = x + _onehot_rows(tidx, typ_ref, 16, blk)
    out_ref[...] = _ln(x, g, b)


def kernel(input_tok, input_tok_type, input_tok_pos, input_ent_tok,
           input_ent_tok_length, input_ent_mask_type, input_ent,
           input_ent_type, ent_candidates, word_emb, ent_emb, pos_emb,
           type_emb, mask_emb, fusion_w, fusion_b, ln_g, ln_b):
    B, T = input_tok.shape
    _, E, M = input_ent_tok.shape
    _, C = ent_candidates.shape
    BT, BE, BC = B * T, B * E, B * C
    f32 = jnp.float32
    bf16 = jnp.bfloat16

    tok_idx = input_tok.reshape(NW, TOK_NC * K)
    et_idx = input_ent_tok.reshape(NW, ET_NC * K)
    ee_idx = jnp.pad(input_ent.reshape(NW, EE_NC * K), ((0, 0), (0, 64)))
    cd_idx = ent_candidates.reshape(NW, CD_NC * K)

    tok_rows = _sc_tok(tok_idx, word_emb, BT)
    et_sum = _sc_et(et_idx, word_emb, BE)

    BLK = 4096
    nb = BT // BLK
    pos256 = pos_emb[:256].astype(bf16)
    typ16 = jnp.zeros((16, H), f32).at[:type_emb.shape[0]].set(type_emb)
    typ16 = typ16.astype(bf16)
    ints1 = jnp.stack([input_tok_pos.reshape(nb, BLK),
                       input_tok_type.reshape(nb, BLK)], axis=1)
    ints1 = jnp.pad(ints1, ((0, 0), (0, 6), (0, 0)))
    g2 = ln_g.reshape(1, H)
    b2 = ln_b.reshape(1, H)

    tok_out = pl.pallas_call(
        functools.partial(_tok_tc_body, blk=BLK),
        grid=(nb,),
        in_specs=[
            pl.BlockSpec((BLK, H), lambda i: (i, 0)),
            pl.BlockSpec((1, 8, BLK), lambda i: (i, 0, 0)),
            pl.BlockSpec((256, H), lambda i: (0, 0)),
            pl.BlockSpec((16, H), lambda i: (0, 0)),
            pl.BlockSpec((1, H), lambda i: (0, 0)),
            pl.BlockSpec((1, H), lambda i: (0, 0)),
        ],
        out_specs=pl.BlockSpec((BLK, H), lambda i: (i, 0)),
        out_shape=jax.ShapeDtypeStruct((BT, H), f32),
    )(tok_rows, ints1, pos256, typ16, g2, b2)

    cd_rows, ee_rows = _sc_cd(cd_idx, ee_idx, ent_emb, BC, BE)

    BLK2 = 6400
    nb2 = BE // BLK2
    ints2 = jnp.stack([input_ent_mask_type.reshape(nb2, BLK2),
                       input_ent_type.reshape(nb2, BLK2),
                       input_ent_tok_length.reshape(nb2, BLK2)], axis=1)
    ints2 = jnp.pad(ints2, ((0, 0), (0, 5), (0, 0)))
    mask8 = jnp.zeros((8, H), f32).at[:mask_emb.shape[0]].set(mask_emb)
    mask8 = mask8.astype(bf16)
    w1 = fusion_w[:H].astype(bf16)
    w2 = fusion_w[H:].astype(bf16)
    fb2 = fusion_b.reshape(1, H)
    recip16 = (1.0 / jnp.maximum(jnp.arange(16, dtype=f32), 1.0)
               ).reshape(16, 1)

    ent_out = pl.pallas_call(
        functools.partial(_ent_tc_body, blk=BLK2),
        grid=(nb2,),
        in_specs=[
            pl.BlockSpec((BLK2, H), lambda i: (i, 0)),
            pl.BlockSpec((BLK2, H), lambda i: (i, 0)),
            pl.BlockSpec((1, 8, BLK2), lambda i: (i, 0, 0)),
            pl.BlockSpec((8, H), lambda i: (0, 0)),
            pl.BlockSpec((H, H), lambda i: (0, 0)),
            pl.BlockSpec((H, H), lambda i: (0, 0)),
            pl.BlockSpec((1, H), lambda i: (0, 0)),
            pl.BlockSpec((16, H), lambda i: (0, 0)),
            pl.BlockSpec((16, 1), lambda i: (0, 0)),
            pl.BlockSpec((1, H), lambda i: (0, 0)),
            pl.BlockSpec((1, H), lambda i: (0, 0)),
        ],
        out_specs=pl.BlockSpec((BLK2, H), lambda i: (i, 0)),
        out_shape=jax.ShapeDtypeStruct((BE, H), f32),
    )(ee_rows, et_sum, ints2, mask8, w1, w2, fb2, typ16, recip16, g2, b2)

    return (tok_out.reshape(B, T, H),
            ent_out.reshape(B, E, H),
            cd_rows.reshape(B, C, H))


# speedup vs baseline: 1.8699x; 1.0124x over previous
import functools

import jax
import jax.numpy as jnp
from jax import lax
from jax.experimental import pallas as pl
from jax.experimental.pallas import tpu as pltpu
from jax.experimental.pallas import tpu_sc as plsc

EPS = 1e-12
NW = 32
H = 128

K = 80
TOK_NC = 80
ET_NC = 200
ET_SEG = 5
EE_NC = 20
CD_NC = 40


def _gather_task(wid, table, idx_buf, out, bufs, sgs, sws, nchunks, rpw):
    base = wid * rpw

    def g_start(c, i):
        pltpu.make_async_copy(table.at[idx_buf.at[pl.ds(c * K, K)]],
                              bufs[i], sgs[i]).start()

    def g_wait(c, i):
        pltpu.make_async_copy(table.at[idx_buf.at[pl.ds(c * K, K)]],
                              bufs[i], sgs[i]).wait()

    def w_start(c, i):
        pltpu.make_async_copy(bufs[i], out.at[pl.ds(base + c * K, K)],
                              sws[i]).start()

    def w_wait(c, i):
        pltpu.make_async_copy(bufs[i], out.at[pl.ds(base + c * K, K)],
                              sws[i]).wait()

    for i in range(3):
        g_start(i, i)

    def quad(q, carry):
        for p in range(4):
            c = 4 * q + p
            pn = (p + 3) % 4
            g_wait(c, p)
            w_start(c, p)

            @pl.when(c > 0)
            def _(c=c, pn=pn):
                w_wait(c - 1, pn)

            @pl.when(c + 3 < nchunks)
            def _(c=c, pn=pn):
                g_start(c + 3, pn)
        return carry

    lax.fori_loop(0, nchunks // 4, quad, 0)
    w_wait(nchunks - 1, 3)


def _sc_tok_body(tok_idx, word, tok_out,
                 tok_ib, b0, b1, b2, b3, s0, s1, s2, s3,
                 w0, w1, w2, w3):
    wid = lax.axis_index("s") * 2 + lax.axis_index("c")
    pltpu.sync_copy(tok_idx.at[wid], tok_ib)
    _gather_task(wid, word, tok_ib, tok_out, (b0, b1, b2, b3),
                 (s0, s1, s2, s3), (w0, w1, w2, w3),
                 TOK_NC, TOK_NC * K)


def _sc_et_body(et_idx, word, et_out,
                et_ib, b0, b1, b2, b3, acc, s0, s1, s2, s3):
    wid = lax.axis_index("s") * 2 + lax.axis_index("c")
    pltpu.sync_copy(et_idx.at[wid], et_ib)
    bufs = (b0, b1, b2, b3)
    sgs = (s0, s1, s2, s3)
    seg_chunks = ET_NC // ET_SEG
    seg_rows = seg_chunks * 8

    def g_start(c, i):
        pltpu.make_async_copy(word.at[et_ib.at[pl.ds(c * K, K)]],
                              bufs[i], sgs[i]).start()

    def g_wait(c, i):
        pltpu.make_async_copy(word.at[et_ib.at[pl.ds(c * K, K)]],
                              bufs[i], sgs[i]).wait()

    def reduce(buf, brow):
        def rbody(r, carry):
            for cc in range(8):
                sl = pl.ds(cc * 16, 16)
                v = buf[10 * r, sl]
                for m in range(1, 10):
                    v = v + buf[10 * r + m, sl]
                acc[brow + r, sl] = v
            return carry
        lax.fori_loop(0, 8, rbody, 0)

    def seg_body(seg, carry):
        ch0 = seg * seg_chunks
        for i in range(3):
            g_start(ch0 + i, i)

        def quad(q, inner):
            for p in range(4):
                lc = 4 * q + p
                g_wait(ch0 + lc, p)
                reduce(bufs[p], lc * 8)

                @pl.when(lc + 3 < seg_chunks)
                def _(lc=lc, p=p):
                    g_start(ch0 + lc + 3, (p + 3) % 4)
            return inner

        lax.fori_loop(0, seg_chunks // 4, quad, 0)
        pltpu.sync_copy(acc,
                        et_out.at[pl.ds(wid * (ET_NC * 8) + seg * seg_rows,
                                        seg_rows)])
        return carry

    lax.fori_loop(0, ET_SEG, seg_body, 0)


def _sc_cd_body(cd_idx, ee_idx, ent, cd_out, ee_out,
                cd_ib, ee_ib, b0, b1, b2, b3,
                s0, s1, s2, s3, w0, w1, w2, w3):
    wid = lax.axis_index("s") * 2 + lax.axis_index("c")
    pltpu.sync_copy(cd_idx.at[wid], cd_ib)
    pltpu.sync_copy(ee_idx.at[wid], ee_ib)
    bufs = (b0, b1, b2, b3)
    sgs = (s0, s1, s2, s3)
    sws = (w0, w1, w2, w3)
    _gather_task(wid, ent, cd_ib, cd_out, bufs, sgs, sws, CD_NC, CD_NC * K)
    _gather_task(wid, ent, ee_ib, ee_out, bufs, sgs, sws, EE_NC, EE_NC * K)


_MESH = dict(core_axis_name="c", subcore_axis_name="s")


def _sems(n):
    return [pltpu.SemaphoreType.DMA] * n


def _bufs(n):
    return [pltpu.VMEM((K, H), jnp.float32)] * n


def _sc_tok(tok_idx, word_emb, bt):
    return pl.kernel(
        _sc_tok_body,
        out_type=jax.ShapeDtypeStruct((bt, H), jnp.float32),
        mesh=plsc.VectorSubcoreMesh(**_MESH),
        scratch_types=[pltpu.VMEM((TOK_NC * K,), jnp.int32)]
        + _bufs(4) + _sems(8),
    )(tok_idx, word_emb)


def _sc_et(et_idx, word_emb, be):
    return pl.kernel(
        _sc_et_body,
        out_type=jax.ShapeDtypeStruct((be, H), jnp.float32),
        mesh=plsc.VectorSubcoreMesh(**_MESH),
        scratch_types=[pltpu.VMEM((ET_NC * K,), jnp.int32)]
        + _bufs(4)
        + [pltpu.VMEM((ET_NC // ET_SEG * 8, H), jnp.float32)] + _sems(4),
    )(et_idx, word_emb)


def _sc_cd(cd_idx, ee_idx, ent_emb, bc, be):
    return pl.kernel(
        _sc_cd_body,
        out_type=[jax.ShapeDtypeStruct((bc, H), jnp.float32),
                  jax.ShapeDtypeStruct((be, H), jnp.float32)],
        mesh=plsc.VectorSubcoreMesh(**_MESH),
        scratch_types=[pltpu.VMEM((CD_NC * K,), jnp.int32),
                       pltpu.VMEM((1664,), jnp.int32)]
        + _bufs(4) + _sems(8),
    )(cd_idx, ee_idx, ent_emb)


def _ln(x, g, b):
    m = jnp.mean(x, axis=-1, keepdims=True)
    v = jnp.mean((x - m) ** 2, axis=-1, keepdims=True)
    return (x - m) / jnp.sqrt(v + EPS) * g + b


_DN = (((0,), (0,)), ((), ()))
_DNR = (((1,), (0,)), ((), ()))


def _onehot(idx_1xn, width, blk, dtype):
    iota = lax.broadcasted_iota(jnp.int32, (width, blk), 0)
    return (iota == idx_1xn).astype(dtype)


def _onehot_rows(idx_1xn, table_ref, width, blk):
    oh = _onehot(idx_1xn, width, blk, jnp.bfloat16)
    return lax.dot_general(oh, table_ref[...], _DN,
                           preferred_element_type=jnp.float32)


def _tok_tc_body(rows_ref, ints_ref, pos_ref, typ_ref, g_ref, b_ref,
                 out_ref, *, blk):
    x = rows_ref[...]
    x = x + _onehot_rows(ints_ref[0, 0:1, :], pos_ref, 256, blk)
    x = x + _onehot_rows(ints_ref[0, 1:2, :], typ_ref, 16, blk)
    out_ref[...] = _ln(x, g_ref[...], b_ref[...])


def _ent_tc_body(ee_ref, es_ref, ints_ref, mask_ref, w1_ref, w2_ref,
                 fb_ref, typ_ref, recip_ref, g_ref, b_ref,
                 out_ref, *, blk):
    g = g_ref[...]
    b = b_ref[...]
    midx = ints_ref[0, 0:1, :]
    tidx = ints_ref[0, 1:2, :]
    lidx = ints_ref[0, 2:3, :]
    oh_len = _onehot(lidx, 16, blk, jnp.float32)
    rlen = lax.dot_general(oh_len, recip_ref[...], _DN,
                           preferred_element_type=jnp.float32)
    oh_mask = _onehot(midx, 8, blk, jnp.float32)
    e0 = (lax.broadcasted_iota(jnp.int32, (8, 1), 0) == 0).astype(jnp.float32)
    mnz0 = lax.dot_general(oh_mask, e0, _DN,
                           preferred_element_type=jnp.float32)
    mrows = _onehot_rows(midx, mask_ref, 8, blk)
    et = mnz0 * (es_ref[...] * rlen) + (1.0 - mnz0) * mrows
    x = lax.dot_general(ee_ref[...].astype(jnp.bfloat16), w1_ref[...], _DNR,
                        preferred_element_type=jnp.float32)
    x = x + lax.dot_general(et.astype(jnp.bfloat16), w2_ref[...], _DNR,
                            preferred_element_type=jnp.float32)
    x = x + fb_ref[...]
    x = 0.5 * x * (1.0 + lax.erf(x * (2.0 ** -0.5)))
    x = _ln(x, g, b)
    x = x + _onehot_rows(tidx, typ_ref, 16, blk)
    out_ref[...] = _ln(x, g, b)


def kernel(input_tok, input_tok_type, input_tok_pos, input_ent_tok,
           input_ent_tok_length, input_ent_mask_type, input_ent,
           input_ent_type, ent_candidates, word_emb, ent_emb, pos_emb,
           type_emb, mask_emb, fusion_w, fusion_b, ln_g, ln_b):
    B, T = input_tok.shape
    _, E, M = input_ent_tok.shape
    _, C = ent_candidates.shape
    BT, BE, BC = B * T, B * E, B * C
    f32 = jnp.float32
    bf16 = jnp.bfloat16

    tok_idx = input_tok.reshape(NW, TOK_NC * K)
    et_idx = input_ent_tok.reshape(NW, ET_NC * K)
    ee_idx = jnp.pad(input_ent.reshape(NW, EE_NC * K), ((0, 0), (0, 64)))
    cd_idx = ent_candidates.reshape(NW, CD_NC * K)

    tok_rows = _sc_tok(tok_idx, word_emb, BT)
    et_sum = _sc_et(et_idx, word_emb, BE)

    BLK = 8192
    nb = BT // BLK
    pos256 = pos_emb[:256].astype(bf16)
    typ16 = jnp.zeros((16, H), f32).at[:type_emb.shape[0]].set(type_emb)
    typ16 = typ16.astype(bf16)
    ints1 = jnp.stack([input_tok_pos.reshape(nb, BLK),
                       input_tok_type.reshape(nb, BLK)], axis=1)
    ints1 = jnp.pad(ints1, ((0, 0), (0, 6), (0, 0)))
    g2 = ln_g.reshape(1, H)
    b2 = ln_b.reshape(1, H)

    tok_out = pl.pallas_call(
        functools.partial(_tok_tc_body, blk=BLK),
        grid=(nb,),
        in_specs=[
            pl.BlockSpec((BLK, H), lambda i: (i, 0)),
            pl.BlockSpec((1, 8, BLK), lambda i: (i, 0, 0)),
            pl.BlockSpec((256, H), lambda i: (0, 0)),
            pl.BlockSpec((16, H), lambda i: (0, 0)),
            pl.BlockSpec((1, H), lambda i: (0, 0)),
            pl.BlockSpec((1, H), lambda i: (0, 0)),
        ],
        out_specs=pl.BlockSpec((BLK, H), lambda i: (i, 0)),
        out_shape=jax.ShapeDtypeStruct((BT, H), f32),
    )(tok_rows, ints1, pos256, typ16, g2, b2)

    cd_rows, ee_rows = _sc_cd(cd_idx, ee_idx, ent_emb, BC, BE)

    BLK2 = 6400
    nb2 = BE // BLK2
    ints2 = jnp.stack([input_ent_mask_type.reshape(nb2, BLK2),
                       input_ent_type.reshape(nb2, BLK2),
                       input_ent_tok_length.reshape(nb2, BLK2)], axis=1)
    ints2 = jnp.pad(ints2, ((0, 0), (0, 5), (0, 0)))
    mask8 = jnp.zeros((8, H), f32).at[:mask_emb.shape[0]].set(mask_emb)
    mask8 = mask8.astype(bf16)
    w1 = fusion_w[:H].astype(bf16)
    w2 = fusion_w[H:].astype(bf16)
    fb2 = fusion_b.reshape(1, H)
    recip16 = (1.0 / jnp.maximum(jnp.arange(16, dtype=f32), 1.0)
               ).reshape(16, 1)

    ent_out = pl.pallas_call(
        functools.partial(_ent_tc_body, blk=BLK2),
        grid=(nb2,),
        in_specs=[
            pl.BlockSpec((BLK2, H), lambda i: (i, 0)),
            pl.BlockSpec((BLK2, H), lambda i: (i, 0)),
            pl.BlockSpec((1, 8, BLK2), lambda i: (i, 0, 0)),
            pl.BlockSpec((8, H), lambda i: (0, 0)),
            pl.BlockSpec((H, H), lambda i: (0, 0)),
            pl.BlockSpec((H, H), lambda i: (0, 0)),
            pl.BlockSpec((1, H), lambda i: (0, 0)),
            pl.BlockSpec((16, H), lambda i: (0, 0)),
            pl.BlockSpec((16, 1), lambda i: (0, 0)),
            pl.BlockSpec((1, H), lambda i: (0, 0)),
            pl.BlockSpec((1, H), lambda i: (0, 0)),
        ],
        out_specs=pl.BlockSpec((BLK2, H), lambda i: (i, 0)),
        out_shape=jax.ShapeDtypeStruct((BE, H), f32),
    )(ee_rows, et_sum, ints2, mask8, w1, w2, fb2, typ16, recip16, g2, b2)

    return (tok_out.reshape(B, T, H),
            ent_out.reshape(B, E, H),
            cd_rows.reshape(B, C, H))
